# baseline (speedup 1.0000x reference)
import functools

import jax
import jax.numpy as jnp
import numpy as np
from jax import lax
from jax.experimental import pallas as pl
from jax.experimental.pallas import tpu as pltpu
from jax.experimental.pallas import tpu_sc as plsc

N_LEVELS = 16
F_PER_LEVEL = 2
LOG2_T = 19
TSIZE = 1 << LOG2_T
MASK = np.uint32(TSIZE - 1)
P1 = np.uint32(2654435761)
P2 = np.uint32(805459861)
NPTS = 262144
WIDTH = 128
NOUT = 16
NIN = N_LEVELS * F_PER_LEVEL

RES = [int(np.floor(16 * 1.5 ** l)) for l in range(N_LEVELS)]
DENSE = [(r + 1) ** 3 <= TSIZE for r in RES]
CORNERS = [(dx, dy, dz) for dx in (0, 1) for dy in (0, 1) for dz in (0, 1)]

NW = 32
PTS_W = NPTS // NW
CHUNK = 128
GROUPS = CHUNK // 16
NGATH = CHUNK * 256
NSTREAM = NGATH // 128
NCHUNK = PTS_W // CHUNK


def _sc_encode(xn_t, tab):
    mesh = plsc.VectorSubcoreMesh(core_axis_name="c", subcore_axis_name="s")

    @functools.partial(
        pl.kernel,
        out_type=jax.ShapeDtypeStruct((NPTS * NIN,), jnp.float32),
        mesh=mesh,
        compiler_params=pltpu.CompilerParams(needs_layout_passes=False),
        scratch_types=[
            pltpu.VMEM((PTS_W,), jnp.float32),
            pltpu.VMEM((PTS_W,), jnp.float32),
            pltpu.VMEM((PTS_W,), jnp.float32),
            pltpu.VMEM((NGATH,), jnp.int32),
            pltpu.VMEM((CHUNK * 128,), jnp.float32),
            pltpu.VMEM((NGATH,), jnp.float32),
            pltpu.VMEM((CHUNK * NIN,), jnp.float32),
            pltpu.SemaphoreType.DMA,
        ],
    )
    def enc(xn_hbm, tab_hbm, pe_hbm, xb0, xb1, xb2, idxb, wtb, rowsb, peb, sem):
        wid = lax.axis_index("s") * 2 + lax.axis_index("c")
        wbase = wid * PTS_W
        for d, xbd in enumerate((xb0, xb1, xb2)):
            pltpu.sync_copy(xn_hbm.at[pl.ds(d * NPTS + wbase, PTS_W)], xbd)

        iota = lax.iota(jnp.int32, 16)
        iota32 = iota * NIN

        def grp_a(g, carry):
            p0 = carry + g * 16
            xv = xb0[pl.ds(p0, 16)]
            yv = xb1[pl.ds(p0, 16)]
            zv = xb2[pl.ds(p0, 16)]
            gr = g * 16
            for l in range(N_LEVELS):
                res = RES[l]
                rf = float(res)
                px = xv * rf
                py = yv * rf
                pz = zv * rf
                cx0 = px.astype(jnp.int32)
                cy0 = py.astype(jnp.int32)
                cz0 = pz.astype(jnp.int32)
                wx = px - cx0.astype(jnp.float32)
                wy = py - cy0.astype(jnp.float32)
                wz = pz - cz0.astype(jnp.float32)
                cx1 = jnp.minimum(cx0 + 1, res)
                cy1 = jnp.minimum(cy0 + 1, res)
                cz1 = jnp.minimum(cz0 + 1, res)
                loff = l << LOG2_T
                if DENSE[l]:
                    r1 = res + 1
                    ax = (cx0, cx1)
                    by = (cy0 * r1, cy1 * r1)
                    tz = (cz0 * (r1 * r1) + loff, cz1 * (r1 * r1) + loff)
                    idx8 = [ax[dx] + by[dy] + tz[dz] for (dx, dy, dz) in CORNERS]
                else:
                    hx = (cx0.astype(jnp.uint32), cx1.astype(jnp.uint32))
                    hy = (cy0.astype(jnp.uint32) * P1, cy1.astype(jnp.uint32) * P1)
                    hz = (cz0.astype(jnp.uint32) * P2, cz1.astype(jnp.uint32) * P2)
                    idx8 = [
                        ((hx[dx] ^ hy[dy] ^ hz[dz]) & MASK).astype(jnp.int32) + loff
                        for (dx, dy, dz) in CORNERS
                    ]
                wx0 = 1.0 - wx
                wy0 = 1.0 - wy
                wz0 = 1.0 - wz
                wxy = ((wx0 * wy0, wx0 * wy), (wx * wy0, wx * wy))
                wzt = (wz0, wz)
                rowb = (gr + l) * 256
                wrow = (gr + l) * 128
                for c, (dx, dy, dz) in enumerate(CORNERS):
                    i2 = idx8[c] * 2
                    idxb[pl.ds(rowb + c * 32, 16)] = i2
                    idxb[pl.ds(rowb + c * 32 + 16, 16)] = i2 + 1
                    wtb[pl.ds(wrow + c * 16, 16)] = wxy[dx][dy] * wzt[dz]
            return carry

        def fire(j, carry):
            pltpu.make_async_copy(
                tab_hbm.at[idxb.at[pl.ds(j * 128, 128)]],
                rowsb.at[pl.ds(j * 128, 128)], sem).start()
            return carry

        def drain(j, carry):
            pltpu.make_async_copy(
                tab_hbm.at[idxb.at[pl.ds(j * 128, 128)]],
                rowsb.at[pl.ds(j * 128, 128)], sem).wait()
            return carry

        def grp_c(g, carry):
            for l in range(N_LEVELS):
                row = g * 16 + l
                rbase = row * 256
                wbase_ = row * 128
                f0 = jnp.zeros((16,), jnp.float32)
                f1 = jnp.zeros((16,), jnp.float32)
                for c in range(8):
                    wt = wtb[pl.ds(wbase_ + c * 16, 16)]
                    f0 = f0 + wt * rowsb[pl.ds(rbase + c * 32, 16)]
                    f1 = f1 + wt * rowsb[pl.ds(rbase + c * 32 + 16, 16)]
                plsc.store_scatter(peb, [iota32 + (g * 16 * NIN + 2 * l)], f0)
                plsc.store_scatter(peb, [iota32 + (g * 16 * NIN + 2 * l + 1)], f1)
            return carry

        def chunk_body(ci, carry):
            cbase = ci * CHUNK
            lax.fori_loop(0, GROUPS, grp_a, cbase)
            lax.fori_loop(0, NSTREAM, fire, 0)
            lax.fori_loop(0, NSTREAM, drain, 0)
            lax.fori_loop(0, GROUPS, grp_c, 0)
            pltpu.sync_copy(peb, pe_hbm.at[pl.ds((wbase + cbase) * NIN, CHUNK * NIN)])
            return carry

        lax.fori_loop(0, NCHUNK, chunk_body, 0)

    return enc(xn_t, tab)


def _mlp(pe, W1, b1, W2, b2, W3, b3):
    B = 1024
    grid = (NPTS // B,)

    def body(pe_ref, w1_ref, b1_ref, w2_ref, b2_ref, w3_ref, b3_ref, z_ref, den_ref):
        dn = (((1,), (0,)), ((), ()))
        h = lax.dot_general(pe_ref[...], w1_ref[...], dn,
                            precision=lax.Precision.HIGHEST)
        h = jnp.maximum(h + b1_ref[...][None, :], 0.0)
        h = lax.dot_general(h, w2_ref[...], dn, precision=lax.Precision.HIGHEST)
        h = jnp.maximum(h + b2_ref[...][None, :], 0.0)
        z = lax.dot_general(h, w3_ref[...], dn, precision=lax.Precision.HIGHEST)
        z = z + b3_ref[...][None, :]
        z_ref[...] = z
        z0 = z[:, 0]
        den_ref[...] = jnp.maximum(z0, 0.0) + jnp.log1p(jnp.exp(-jnp.abs(z0)))

    return pl.pallas_call(
        body,
        grid=grid,
        in_specs=[
            pl.BlockSpec((B, NIN), lambda i: (i, 0)),
            pl.BlockSpec((NIN, WIDTH), lambda i: (0, 0)),
            pl.BlockSpec((WIDTH,), lambda i: (0,)),
            pl.BlockSpec((WIDTH, WIDTH), lambda i: (0, 0)),
            pl.BlockSpec((WIDTH,), lambda i: (0,)),
            pl.BlockSpec((WIDTH, NOUT), lambda i: (0, 0)),
            pl.BlockSpec((NOUT,), lambda i: (0,)),
        ],
        out_specs=[
            pl.BlockSpec((B, NOUT), lambda i: (i, 0)),
            pl.BlockSpec((B,), lambda i: (i,)),
        ],
        out_shape=[
            jax.ShapeDtypeStruct((NPTS, NOUT), jnp.float32),
            jax.ShapeDtypeStruct((NPTS,), jnp.float32),
        ],
    )(pe, W1, b1, W2, b2, W3, b3)


def kernel(x, bounding_box, table, W1, b1, W2, b2, W3, b3):
    xn = (x - bounding_box[0]) / (bounding_box[1] - bounding_box[0])
    xn_t = xn.T.reshape(-1)
    tab = table.reshape(-1)
    pe = _sc_encode(xn_t, tab).reshape(NPTS, NIN)
    z, density = _mlp(pe, W1, b1, W2, b2, W3, b3)
    return (density, pe, z)


# speedup vs baseline: 1.0297x; 1.0297x over previous
import functools

import jax
import jax.numpy as jnp
import numpy as np
from jax import lax
from jax.experimental import pallas as pl
from jax.experimental.pallas import tpu as pltpu
from jax.experimental.pallas import tpu_sc as plsc

N_LEVELS = 16
F_PER_LEVEL = 2
LOG2_T = 19
TSIZE = 1 << LOG2_T
MASK = np.uint32(TSIZE - 1)
P1 = np.uint32(2654435761)
P2 = np.uint32(805459861)
NPTS = 262144
WIDTH = 128
NOUT = 16
NIN = N_LEVELS * F_PER_LEVEL

RES = [int(np.floor(16 * 1.5 ** l)) for l in range(N_LEVELS)]
DENSE = [(r + 1) ** 3 <= TSIZE for r in RES]
CORNERS = [(dx, dy, dz) for dx in (0, 1) for dy in (0, 1) for dz in (0, 1)]

NW = 32
PTS_W = NPTS // NW
CHUNK = 128
GROUPS = CHUNK // 16
NGATH = CHUNK * 256
NSTREAM = NGATH // 128
NCHUNK = PTS_W // CHUNK


def _sc_encode(xn_t, tab):
    mesh = plsc.VectorSubcoreMesh(core_axis_name="c", subcore_axis_name="s")

    @functools.partial(
        pl.kernel,
        out_type=jax.ShapeDtypeStruct((NPTS * NIN,), jnp.float32),
        mesh=mesh,
        compiler_params=pltpu.CompilerParams(needs_layout_passes=False),
        scratch_types=[
            pltpu.VMEM((PTS_W,), jnp.float32),
            pltpu.VMEM((PTS_W,), jnp.float32),
            pltpu.VMEM((PTS_W,), jnp.float32),
            pltpu.VMEM((NGATH,), jnp.int32),
            pltpu.VMEM((CHUNK * 128,), jnp.float32),
            pltpu.VMEM((NGATH,), jnp.float32),
            pltpu.VMEM((CHUNK * NIN,), jnp.float32),
            pltpu.SemaphoreType.DMA,
        ],
    )
    def enc(xn_hbm, tab_hbm, pe_hbm, xb0, xb1, xb2, idxb, wtb, rowsb, peb, sem):
        wid = lax.axis_index("s") * 2 + lax.axis_index("c")
        wbase = wid * PTS_W
        for d, xbd in enumerate((xb0, xb1, xb2)):
            pltpu.sync_copy(xn_hbm.at[pl.ds(d * NPTS + wbase, PTS_W)], xbd)

        iota = lax.iota(jnp.int32, 16)
        iota32 = iota * NIN

        def grp_a(g, carry):
            p0 = carry + g * 16
            xv = xb0[pl.ds(p0, 16)]
            yv = xb1[pl.ds(p0, 16)]
            zv = xb2[pl.ds(p0, 16)]
            gr = g * 16
            for l in range(N_LEVELS):
                res = RES[l]
                rf = float(res)
                px = xv * rf
                py = yv * rf
                pz = zv * rf
                cx0 = px.astype(jnp.int32)
                cy0 = py.astype(jnp.int32)
                cz0 = pz.astype(jnp.int32)
                wx = px - cx0.astype(jnp.float32)
                wy = py - cy0.astype(jnp.float32)
                wz = pz - cz0.astype(jnp.float32)
                cx1 = jnp.minimum(cx0 + 1, res)
                cy1 = jnp.minimum(cy0 + 1, res)
                cz1 = jnp.minimum(cz0 + 1, res)
                loff = l << LOG2_T
                if DENSE[l]:
                    r1 = res + 1
                    ax = (cx0, cx1)
                    by = (cy0 * r1, cy1 * r1)
                    tz = (cz0 * (r1 * r1) + loff, cz1 * (r1 * r1) + loff)
                    idx8 = [ax[dx] + by[dy] + tz[dz] for (dx, dy, dz) in CORNERS]
                else:
                    hx = (cx0.astype(jnp.uint32), cx1.astype(jnp.uint32))
                    hy = (cy0.astype(jnp.uint32) * P1, cy1.astype(jnp.uint32) * P1)
                    hz = (cz0.astype(jnp.uint32) * P2, cz1.astype(jnp.uint32) * P2)
                    idx8 = [
                        ((hx[dx] ^ hy[dy] ^ hz[dz]) & MASK).astype(jnp.int32) + loff
                        for (dx, dy, dz) in CORNERS
                    ]
                wx0 = 1.0 - wx
                wy0 = 1.0 - wy
                wz0 = 1.0 - wz
                wxy = ((wx0 * wy0, wx0 * wy), (wx * wy0, wx * wy))
                wzt = (wz0, wz)
                rowb = (gr + l) * 256
                wrow = (gr + l) * 128
                for c, (dx, dy, dz) in enumerate(CORNERS):
                    i2 = idx8[c] * 2
                    idxb[pl.ds(rowb + c * 32, 16)] = i2
                    idxb[pl.ds(rowb + c * 32 + 16, 16)] = i2 + 1
                    wtb[pl.ds(wrow + c * 16, 16)] = wxy[dx][dy] * wzt[dz]
            return carry

        def grp_c(g, carry):
            for l in range(N_LEVELS):
                row = g * 16 + l
                rbase = row * 256
                wbase_ = row * 128
                f0 = jnp.zeros((16,), jnp.float32)
                f1 = jnp.zeros((16,), jnp.float32)
                for c in range(8):
                    wt = wtb[pl.ds(wbase_ + c * 16, 16)]
                    f0 = f0 + wt * rowsb[pl.ds(rbase + c * 32, 16)]
                    f1 = f1 + wt * rowsb[pl.ds(rbase + c * 32 + 16, 16)]
                plsc.store_scatter(peb, [iota32 + (g * 16 * NIN + 2 * l)], f0)
                plsc.store_scatter(peb, [iota32 + (g * 16 * NIN + 2 * l + 1)], f1)
            return carry

        def chunk_body(ci, carry):
            cbase = ci * CHUNK
            lax.fori_loop(0, GROUPS, grp_a, cbase)
            cp = pltpu.make_async_copy(tab_hbm.at[idxb], rowsb, sem)
            cp.start()
            cp.wait()
            lax.fori_loop(0, GROUPS, grp_c, 0)
            pltpu.sync_copy(peb, pe_hbm.at[pl.ds((wbase + cbase) * NIN, CHUNK * NIN)])
            return carry

        lax.fori_loop(0, NCHUNK, chunk_body, 0)

    return enc(xn_t, tab)


def _mlp(pe, W1, b1, W2, b2, W3, b3):
    B = 1024
    grid = (NPTS // B,)

    def body(pe_ref, w1_ref, b1_ref, w2_ref, b2_ref, w3_ref, b3_ref, z_ref, den_ref):
        dn = (((1,), (0,)), ((), ()))
        h = lax.dot_general(pe_ref[...], w1_ref[...], dn,
                            preferred_element_type=jnp.float32)
        h = jnp.maximum(h + b1_ref[...][None, :], 0.0)
        h = lax.dot_general(h, w2_ref[...], dn,
                            preferred_element_type=jnp.float32)
        h = jnp.maximum(h + b2_ref[...][None, :], 0.0)
        z = lax.dot_general(h, w3_ref[...], dn,
                            preferred_element_type=jnp.float32)
        z = z + b3_ref[...][None, :]
        z_ref[...] = z
        z0 = z[:, 0]
        den_ref[...] = jnp.maximum(z0, 0.0) + jnp.log(1.0 + jnp.exp(-jnp.abs(z0)))

    return pl.pallas_call(
        body,
        grid=grid,
        in_specs=[
            pl.BlockSpec((B, NIN), lambda i: (i, 0)),
            pl.BlockSpec((NIN, WIDTH), lambda i: (0, 0)),
            pl.BlockSpec((WIDTH,), lambda i: (0,)),
            pl.BlockSpec((WIDTH, WIDTH), lambda i: (0, 0)),
            pl.BlockSpec((WIDTH,), lambda i: (0,)),
            pl.BlockSpec((WIDTH, NOUT), lambda i: (0, 0)),
            pl.BlockSpec((NOUT,), lambda i: (0,)),
        ],
        out_specs=[
            pl.BlockSpec((B, NOUT), lambda i: (i, 0)),
            pl.BlockSpec((B,), lambda i: (i,)),
        ],
        out_shape=[
            jax.ShapeDtypeStruct((NPTS, NOUT), jnp.float32),
            jax.ShapeDtypeStruct((NPTS,), jnp.float32),
        ],
    )(pe, W1, b1, W2, b2, W3, b3)


def kernel(x, bounding_box, table, W1, b1, W2, b2, W3, b3):
    xn = (x - bounding_box[0]) / (bounding_box[1] - bounding_box[0])
    xn_t = xn.T.reshape(-1)
    tab = table.reshape(-1)
    pe = _sc_encode(xn_t, tab).reshape(NPTS, NIN)
    z, density = _mlp(pe, W1, b1, W2, b2, W3, b3)
    return (density, pe, z)


# speedup vs baseline: 1.1567x; 1.1233x over previous
import functools

import jax
import jax.numpy as jnp
import numpy as np
from jax import lax
from jax.experimental import pallas as pl
from jax.experimental.pallas import tpu as pltpu
from jax.experimental.pallas import tpu_sc as plsc

N_LEVELS = 16
F_PER_LEVEL = 2
LOG2_T = 19
TSIZE = 1 << LOG2_T
MASK = np.uint32(TSIZE - 1)
P1 = np.uint32(2654435761)
P2 = np.uint32(805459861)
NPTS = 262144
WIDTH = 128
NOUT = 16
NIN = N_LEVELS * F_PER_LEVEL

RES = [int(np.floor(16 * 1.5 ** l)) for l in range(N_LEVELS)]
DENSE = [(r + 1) ** 3 <= TSIZE for r in RES]
CORNERS = [(dx, dy, dz) for dx in (0, 1) for dy in (0, 1) for dz in (0, 1)]

NW = 32
PTS_W = NPTS // NW
CHUNK = 32
GROUPS = CHUNK // 16
NIDX = CHUNK * 128
ROWW = 8
NCHUNK = PTS_W // CHUNK
NPAIR = NCHUNK // 2


def _sc_encode(xn_t, tab):
    mesh = plsc.VectorSubcoreMesh(core_axis_name="c", subcore_axis_name="s")

    @functools.partial(
        pl.kernel,
        out_type=jax.ShapeDtypeStruct((NPTS * NIN,), jnp.float32),
        mesh=mesh,
        compiler_params=pltpu.CompilerParams(
            needs_layout_passes=False, use_tc_tiling_on_sc=False),
        scratch_types=[
            pltpu.VMEM((PTS_W * 3,), jnp.float32),
            pltpu.VMEM((NIDX,), jnp.int32),
            pltpu.VMEM((NIDX,), jnp.int32),
            pltpu.VMEM((NIDX,), jnp.int32),
            pltpu.VMEM((NIDX,), jnp.int32),
            pltpu.VMEM((NIDX,), jnp.float32),
            pltpu.VMEM((NIDX,), jnp.float32),
            pltpu.VMEM((NIDX, ROWW), jnp.float32),
            pltpu.VMEM((NIDX, ROWW), jnp.float32),
            pltpu.VMEM((CHUNK * NIN,), jnp.float32),
            pltpu.SemaphoreType.DMA,
            pltpu.SemaphoreType.DMA,
        ],
    )
    def enc(xn_hbm, tab_hbm, pe_hbm, xb, idxA, idxB, colA, colB,
            wtA, wtB, rowsA, rowsB, peb, semA, semB):
        wid = lax.axis_index("s") * 2 + lax.axis_index("c")
        wbase = wid * PTS_W
        pltpu.sync_copy(xn_hbm.at[pl.ds(wbase * 3, PTS_W * 3)], xb)

        iota = lax.iota(jnp.int32, 16)
        iota32 = iota * NIN
        iota3 = iota * 3

        def make_grp_a(idxb, colb, wtb):
          def grp_a(g, carry):
            p0 = carry + g * 16
            p3 = iota3 + p0 * 3
            xv = plsc.load_gather(xb, [p3])
            yv = plsc.load_gather(xb, [p3 + 1])
            zv = plsc.load_gather(xb, [p3 + 2])
            gr = g * 16
            for l in range(N_LEVELS):
                res = RES[l]
                rf = float(res)
                px = xv * rf
                py = yv * rf
                pz = zv * rf
                cx0 = px.astype(jnp.int32)
                cy0 = py.astype(jnp.int32)
                cz0 = pz.astype(jnp.int32)
                wx = px - cx0.astype(jnp.float32)
                wy = py - cy0.astype(jnp.float32)
                wz = pz - cz0.astype(jnp.float32)
                cx1 = jnp.minimum(cx0 + 1, res)
                cy1 = jnp.minimum(cy0 + 1, res)
                cz1 = jnp.minimum(cz0 + 1, res)
                loff = l << LOG2_T
                if DENSE[l]:
                    r1 = res + 1
                    ax = (cx0, cx1)
                    by = (cy0 * r1, cy1 * r1)
                    tz = (cz0 * (r1 * r1) + loff, cz1 * (r1 * r1) + loff)
                    idx8 = [ax[dx] + by[dy] + tz[dz] for (dx, dy, dz) in CORNERS]
                else:
                    hx = (cx0.astype(jnp.uint32), cx1.astype(jnp.uint32))
                    hy = (cy0.astype(jnp.uint32) * P1, cy1.astype(jnp.uint32) * P1)
                    hz = (cz0.astype(jnp.uint32) * P2, cz1.astype(jnp.uint32) * P2)
                    idx8 = [
                        ((hx[dx] ^ hy[dy] ^ hz[dz]) & MASK).astype(jnp.int32) + loff
                        for (dx, dy, dz) in CORNERS
                    ]
                wx0 = 1.0 - wx
                wy0 = 1.0 - wy
                wz0 = 1.0 - wz
                wxy = ((wx0 * wy0, wx0 * wy), (wx * wy0, wx * wy))
                wzt = (wz0, wz)
                wrow = (gr + l) * 128
                for c, (dx, dy, dz) in enumerate(CORNERS):
                    p = idx8[c]
                    sl = pl.ds(wrow + c * 16, 16)
                    idxb[sl] = lax.shift_right_logical(p, 2)
                    colb[sl] = (p & 3) * 2
                    wtb[sl] = wxy[dx][dy] * wzt[dz]
            return carry
          return grp_a

        def make_grp_c(colb, wtb, rowsb):
          def grp_c(g, carry):
            for l in range(N_LEVELS):
                row = g * 16 + l
                wbase_ = row * 128
                f0 = jnp.zeros((16,), jnp.float32)
                f1 = jnp.zeros((16,), jnp.float32)
                for c in range(8):
                    sl = pl.ds(wbase_ + c * 16, 16)
                    wt = wtb[sl]
                    cv = colb[sl]
                    rv = iota + (wbase_ + c * 16)
                    d0 = plsc.load_gather(rowsb, [rv, cv])
                    d1 = plsc.load_gather(rowsb, [rv, cv + 1])
                    f0 = f0 + wt * d0
                    f1 = f1 + wt * d1
                plsc.store_scatter(peb, [iota32 + (g * 16 * NIN + 2 * l)], f0)
                plsc.store_scatter(peb, [iota32 + (g * 16 * NIN + 2 * l + 1)], f1)
            return carry
          return grp_c

        grp_a_A = make_grp_a(idxA, colA, wtA)
        grp_a_B = make_grp_a(idxB, colB, wtB)
        grp_c_A = make_grp_c(colA, wtA, rowsA)
        grp_c_B = make_grp_c(colB, wtB, rowsB)

        def copy_A():
            return pltpu.make_async_copy(tab_hbm.at[idxA], rowsA, semA)

        def copy_B():
            return pltpu.make_async_copy(tab_hbm.at[idxB], rowsB, semB)

        def out_pe(cbase):
            pltpu.sync_copy(peb, pe_hbm.at[pl.ds((wbase + cbase) * NIN, CHUNK * NIN)])

        lax.fori_loop(0, GROUPS, grp_a_A, 0)
        copy_A().start()

        def pair_body(k, carry):
            c0 = (2 * k) * CHUNK
            lax.fori_loop(0, GROUPS, grp_a_B, c0 + CHUNK)
            copy_B().start()
            copy_A().wait()
            lax.fori_loop(0, GROUPS, grp_c_A, 0)
            out_pe(c0)

            @pl.when(k < NPAIR - 1)
            def _():
                lax.fori_loop(0, GROUPS, grp_a_A, c0 + 2 * CHUNK)
                copy_A().start()

            copy_B().wait()
            lax.fori_loop(0, GROUPS, grp_c_B, 0)
            out_pe(c0 + CHUNK)
            return carry

        lax.fori_loop(0, NPAIR, pair_body, 0)

    return enc(xn_t, tab)


def _mlp(pe, W1, b1, W2, b2, W3, b3):
    B = 1024
    grid = (NPTS // B,)

    def body(pe_ref, w1_ref, b1_ref, w2_ref, b2_ref, w3_ref, b3_ref, z_ref, den_ref):
        dn = (((1,), (0,)), ((), ()))
        h = lax.dot_general(pe_ref[...], w1_ref[...], dn,
                            preferred_element_type=jnp.float32)
        h = jnp.maximum(h + b1_ref[...][None, :], 0.0)
        h = lax.dot_general(h, w2_ref[...], dn,
                            preferred_element_type=jnp.float32)
        h = jnp.maximum(h + b2_ref[...][None, :], 0.0)
        z = lax.dot_general(h, w3_ref[...], dn,
                            preferred_element_type=jnp.float32)
        z = z + b3_ref[...][None, :]
        z_ref[...] = z
        z0 = z[:, 0]
        den_ref[...] = jnp.maximum(z0, 0.0) + jnp.log(1.0 + jnp.exp(-jnp.abs(z0)))

    return pl.pallas_call(
        body,
        grid=grid,
        in_specs=[
            pl.BlockSpec((B, NIN), lambda i: (i, 0)),
            pl.BlockSpec((NIN, WIDTH), lambda i: (0, 0)),
            pl.BlockSpec((WIDTH,), lambda i: (0,)),
            pl.BlockSpec((WIDTH, WIDTH), lambda i: (0, 0)),
            pl.BlockSpec((WIDTH,), lambda i: (0,)),
            pl.BlockSpec((WIDTH, NOUT), lambda i: (0, 0)),
            pl.BlockSpec((NOUT,), lambda i: (0,)),
        ],
        out_specs=[
            pl.BlockSpec((B, NOUT), lambda i: (i, 0)),
            pl.BlockSpec((B,), lambda i: (i,)),
        ],
        out_shape=[
            jax.ShapeDtypeStruct((NPTS, NOUT), jnp.float32),
            jax.ShapeDtypeStruct((NPTS,), jnp.float32),
        ],
    )(pe, W1, b1, W2, b2, W3, b3)


def kernel(x, bounding_box, table, W1, b1, W2, b2, W3, b3):
    xn = (x - bounding_box[0]) / (bounding_box[1] - bounding_box[0])
    tab = table.reshape(-1, ROWW)
    pe = _sc_encode(xn.reshape(-1), tab).reshape(NPTS, NIN)
    z, density = _mlp(pe, W1, b1, W2, b2, W3, b3)
    return (density, pe, z)


# speedup vs baseline: 4.4014x; 3.8049x over previous
import functools

import jax
import jax.numpy as jnp
import numpy as np
from jax import lax
from jax.experimental import pallas as pl
from jax.experimental.pallas import tpu as pltpu
from jax.experimental.pallas import tpu_sc as plsc

N_LEVELS = 16
F_PER_LEVEL = 2
LOG2_T = 19
TSIZE = 1 << LOG2_T
MASK = np.uint32(TSIZE - 1)
P1 = np.uint32(2654435761)
P2 = np.uint32(805459861)
NPTS = 262144
WIDTH = 128
NOUT = 16
NIN = N_LEVELS * F_PER_LEVEL

RES = [int(np.floor(16 * 1.5 ** l)) for l in range(N_LEVELS)]
DENSE = [(r + 1) ** 3 <= TSIZE for r in RES]
CORNERS = [(dx, dy, dz) for dx in (0, 1) for dy in (0, 1) for dz in (0, 1)]

NW = 32
PTS_W = NPTS // NW
CHUNK = 64
GROUPS = CHUNK // 16
NIDX = CHUNK * 256
NCHUNK = PTS_W // CHUNK
NPAIR = NCHUNK // 2


def _sc_encode(xn_t, tab):
    mesh = plsc.VectorSubcoreMesh(core_axis_name="c", subcore_axis_name="s")

    @functools.partial(
        pl.kernel,
        out_type=jax.ShapeDtypeStruct((NPTS * NIN,), jnp.float32),
        mesh=mesh,
        compiler_params=pltpu.CompilerParams(
            needs_layout_passes=False, use_tc_tiling_on_sc=False),
        scratch_types=[
            pltpu.VMEM((PTS_W * 3,), jnp.float32),
            pltpu.VMEM((NIDX,), jnp.int32),
            pltpu.VMEM((NIDX,), jnp.int32),
            pltpu.VMEM((CHUNK * 128,), jnp.float32),
            pltpu.VMEM((CHUNK * 128,), jnp.float32),
            pltpu.VMEM((NIDX,), jnp.float32),
            pltpu.VMEM((NIDX,), jnp.float32),
            pltpu.VMEM((CHUNK * NIN,), jnp.float32),
            pltpu.SemaphoreType.DMA,
            pltpu.SemaphoreType.DMA,
        ],
    )
    def enc(xn_hbm, tab_hbm, pe_hbm, xb, idxA, idxB,
            wtA, wtB, rowsA, rowsB, peb, semA, semB):
        wid = lax.axis_index("s") * 2 + lax.axis_index("c")
        wbase = wid * PTS_W
        pltpu.sync_copy(xn_hbm.at[pl.ds(wbase * 3, PTS_W * 3)], xb)

        iota = lax.iota(jnp.int32, 16)
        iota32 = iota * NIN
        iota3 = iota * 3

        def make_grp_a(idxb, wtb):
          def grp_a(g, carry):
            p0 = carry + g * 16
            p3 = iota3 + p0 * 3
            xv = plsc.load_gather(xb, [p3])
            yv = plsc.load_gather(xb, [p3 + 1])
            zv = plsc.load_gather(xb, [p3 + 2])
            gr = g * 16
            for l in range(N_LEVELS):
                res = RES[l]
                rf = float(res)
                px = xv * rf
                py = yv * rf
                pz = zv * rf
                cx0 = px.astype(jnp.int32)
                cy0 = py.astype(jnp.int32)
                cz0 = pz.astype(jnp.int32)
                wx = px - cx0.astype(jnp.float32)
                wy = py - cy0.astype(jnp.float32)
                wz = pz - cz0.astype(jnp.float32)
                cx1 = jnp.minimum(cx0 + 1, res)
                cy1 = jnp.minimum(cy0 + 1, res)
                cz1 = jnp.minimum(cz0 + 1, res)
                if DENSE[l]:
                    r1 = res + 1
                    ax = (cx0, cx1)
                    by = (cy0 * r1, cy1 * r1)
                    tz = (cz0 * (r1 * r1), cz1 * (r1 * r1))
                    idx8 = [ax[dx] + by[dy] + tz[dz] for (dx, dy, dz) in CORNERS]
                else:
                    hx = (cx0.astype(jnp.uint32), cx1.astype(jnp.uint32))
                    hy = (cy0.astype(jnp.uint32) * P1, cy1.astype(jnp.uint32) * P1)
                    hz = (cz0.astype(jnp.uint32) * P2, cz1.astype(jnp.uint32) * P2)
                    idx8 = [
                        ((hx[dx] ^ hy[dy] ^ hz[dz]) & MASK).astype(jnp.int32)
                        for (dx, dy, dz) in CORNERS
                    ]
                wx0 = 1.0 - wx
                wy0 = 1.0 - wy
                wz0 = 1.0 - wz
                wxy = ((wx0 * wy0, wx0 * wy), (wx * wy0, wx * wy))
                wzt = (wz0, wz)
                loff2 = l << (LOG2_T + 1)
                wrow = (gr + l) * 128
                rowb = (gr + l) * 256
                for c, (dx, dy, dz) in enumerate(CORNERS):
                    e = idx8[c]
                    p0 = (((e & 0x7FF80) << 1) | (e & 127)) + loff2
                    idxb[pl.ds(rowb + c * 16, 16)] = p0
                    idxb[pl.ds(rowb + 128 + c * 16, 16)] = p0 + 128
                    wtb[pl.ds(wrow + c * 16, 16)] = wxy[dx][dy] * wzt[dz]
            return carry
          return grp_a

        def make_grp_c(wtb, rowsb):
          def grp_c(g, carry):
            for l in range(N_LEVELS):
                row = g * 16 + l
                rbase = row * 256
                wbase_ = row * 128
                f0 = jnp.zeros((16,), jnp.float32)
                f1 = jnp.zeros((16,), jnp.float32)
                for c in range(8):
                    wt = wtb[pl.ds(wbase_ + c * 16, 16)]
                    f0 = f0 + wt * rowsb[pl.ds(rbase + c * 16, 16)]
                    f1 = f1 + wt * rowsb[pl.ds(rbase + 128 + c * 16, 16)]
                plsc.store_scatter(peb, [iota32 + (g * 16 * NIN + 2 * l)], f0)
                plsc.store_scatter(peb, [iota32 + (g * 16 * NIN + 2 * l + 1)], f1)
            return carry
          return grp_c

        grp_a_A = make_grp_a(idxA, wtA)
        grp_a_B = make_grp_a(idxB, wtB)
        grp_c_A = make_grp_c(wtA, rowsA)
        grp_c_B = make_grp_c(wtB, rowsB)

        def copy_A():
            return pltpu.make_async_copy(tab_hbm.at[idxA], rowsA, semA)

        def copy_B():
            return pltpu.make_async_copy(tab_hbm.at[idxB], rowsB, semB)

        def out_pe(cbase):
            pltpu.sync_copy(peb, pe_hbm.at[pl.ds((wbase + cbase) * NIN, CHUNK * NIN)])

        lax.fori_loop(0, GROUPS, grp_a_A, 0)
        copy_A().start()

        def pair_body(k, carry):
            c0 = (2 * k) * CHUNK
            lax.fori_loop(0, GROUPS, grp_a_B, c0 + CHUNK)
            copy_B().start()
            copy_A().wait()
            lax.fori_loop(0, GROUPS, grp_c_A, 0)
            out_pe(c0)

            @pl.when(k < NPAIR - 1)
            def _():
                lax.fori_loop(0, GROUPS, grp_a_A, c0 + 2 * CHUNK)
                copy_A().start()

            copy_B().wait()
            lax.fori_loop(0, GROUPS, grp_c_B, 0)
            out_pe(c0 + CHUNK)
            return carry

        lax.fori_loop(0, NPAIR, pair_body, 0)

    return enc(xn_t, tab)


def _mlp(pe, W1, b1, W2, b2, W3, b3):
    B = 1024
    grid = (NPTS // B,)

    def body(pe_ref, w1_ref, b1_ref, w2_ref, b2_ref, w3_ref, b3_ref, z_ref, den_ref):
        dn = (((1,), (0,)), ((), ()))
        h = lax.dot_general(pe_ref[...], w1_ref[...], dn,
                            preferred_element_type=jnp.float32)
        h = jnp.maximum(h + b1_ref[...][None, :], 0.0)
        h = lax.dot_general(h, w2_ref[...], dn,
                            preferred_element_type=jnp.float32)
        h = jnp.maximum(h + b2_ref[...][None, :], 0.0)
        z = lax.dot_general(h, w3_ref[...], dn,
                            preferred_element_type=jnp.float32)
        z = z + b3_ref[...][None, :]
        z_ref[...] = z
        z0 = z[:, 0]
        den_ref[...] = jnp.maximum(z0, 0.0) + jnp.log(1.0 + jnp.exp(-jnp.abs(z0)))

    return pl.pallas_call(
        body,
        grid=grid,
        in_specs=[
            pl.BlockSpec((B, NIN), lambda i: (i, 0)),
            pl.BlockSpec((NIN, WIDTH), lambda i: (0, 0)),
            pl.BlockSpec((WIDTH,), lambda i: (0,)),
            pl.BlockSpec((WIDTH, WIDTH), lambda i: (0, 0)),
            pl.BlockSpec((WIDTH,), lambda i: (0,)),
            pl.BlockSpec((WIDTH, NOUT), lambda i: (0, 0)),
            pl.BlockSpec((NOUT,), lambda i: (0,)),
        ],
        out_specs=[
            pl.BlockSpec((B, NOUT), lambda i: (i, 0)),
            pl.BlockSpec((B,), lambda i: (i,)),
        ],
        out_shape=[
            jax.ShapeDtypeStruct((NPTS, NOUT), jnp.float32),
            jax.ShapeDtypeStruct((NPTS,), jnp.float32),
        ],
    )(pe, W1, b1, W2, b2, W3, b3)


def kernel(x, bounding_box, table, W1, b1, W2, b2, W3, b3):
    xn = (x - bounding_box[0]) / (bounding_box[1] - bounding_box[0])
    tab = (table.reshape(16, 4096, 128, 2).transpose(0, 1, 3, 2)
           .reshape(-1))
    pe = _sc_encode(xn.reshape(-1), tab).reshape(NPTS, NIN)
    z, density = _mlp(pe, W1, b1, W2, b2, W3, b3)
    return (density, pe, z)


# speedup vs baseline: 7.4180x; 1.6854x over previous
import functools

import jax
import jax.numpy as jnp
import numpy as np
from jax import lax
from jax.experimental import pallas as pl
from jax.experimental.pallas import tpu as pltpu
from jax.experimental.pallas import tpu_sc as plsc

N_LEVELS = 16
F_PER_LEVEL = 2
LOG2_T = 19
TSIZE = 1 << LOG2_T
MASK = np.uint32(TSIZE - 1)
P1 = np.uint32(2654435761)
P2 = np.uint32(805459861)
NPTS = 262144
WIDTH = 128
NOUT = 16
NIN = N_LEVELS * F_PER_LEVEL

RES = [int(np.floor(16 * 1.5 ** l)) for l in range(N_LEVELS)]
DENSE = [(r + 1) ** 3 <= TSIZE for r in RES]
CORNERS = [(dx, dy, dz) for dx in (0, 1) for dy in (0, 1) for dz in (0, 1)]

NW = 32
PTS_W = NPTS // NW
CHUNK = 32
GROUPS = CHUNK // 16
NIDX = CHUNK * 128
ROWW = 8
NCHUNK = PTS_W // CHUNK
NPAIR = NCHUNK // 2
TABF = N_LEVELS * TSIZE * F_PER_LEVEL
RL_CHUNK = 16384


def _sc_relayout(tab_p):
    mesh = plsc.VectorSubcoreMesh(core_axis_name="c", subcore_axis_name="s")
    span = TABF // NW

    @functools.partial(
        pl.kernel,
        out_type=jax.ShapeDtypeStruct((TABF,), jnp.float32),
        mesh=mesh,
        compiler_params=pltpu.CompilerParams(
            needs_layout_passes=False, use_tc_tiling_on_sc=False),
        scratch_types=[
            pltpu.VMEM((RL_CHUNK,), jnp.float32),
            pltpu.VMEM((RL_CHUNK,), jnp.float32),
        ],
    )
    def rl(tab_hbm, out_hbm, inb, outb):
        wid = lax.axis_index("s") * 2 + lax.axis_index("c")
        base = wid * span
        iota2 = lax.iota(jnp.int32, 16) * 2

        def tile_body(t, carry):
            tb = t * 256
            for eb in range(8):
                v0 = inb[pl.ds(tb + eb * 16, 16)]
                v1 = inb[pl.ds(tb + 128 + eb * 16, 16)]
                pos = iota2 + (tb + eb * 32)
                plsc.store_scatter(outb, [pos], v0)
                plsc.store_scatter(outb, [pos + 1], v1)
            return carry

        def chunk_body(ci, carry):
            off = base + ci * RL_CHUNK
            pltpu.sync_copy(tab_hbm.at[pl.ds(off, RL_CHUNK)], inb)
            lax.fori_loop(0, RL_CHUNK // 256, tile_body, 0)
            pltpu.sync_copy(outb, out_hbm.at[pl.ds(off, RL_CHUNK)])
            return carry

        lax.fori_loop(0, span // RL_CHUNK, chunk_body, 0)

    return rl(tab_p)


def _sc_encode(xn_t, tab):
    mesh = plsc.VectorSubcoreMesh(core_axis_name="c", subcore_axis_name="s")

    @functools.partial(
        pl.kernel,
        out_type=jax.ShapeDtypeStruct((NPTS * NIN,), jnp.float32),
        mesh=mesh,
        compiler_params=pltpu.CompilerParams(
            needs_layout_passes=False, use_tc_tiling_on_sc=False),
        scratch_types=[
            pltpu.VMEM((PTS_W * 3,), jnp.float32),
            pltpu.VMEM((NIDX,), jnp.int32),
            pltpu.VMEM((NIDX,), jnp.int32),
            pltpu.VMEM((NIDX,), jnp.int32),
            pltpu.VMEM((NIDX,), jnp.int32),
            pltpu.VMEM((NIDX,), jnp.float32),
            pltpu.VMEM((NIDX,), jnp.float32),
            pltpu.VMEM((NIDX, ROWW), jnp.float32),
            pltpu.VMEM((NIDX, ROWW), jnp.float32),
            pltpu.VMEM((CHUNK * NIN,), jnp.float32),
            pltpu.SemaphoreType.DMA,
            pltpu.SemaphoreType.DMA,
        ],
    )
    def enc(xn_hbm, tab_hbm, pe_hbm, xb, idxA, idxB, colA, colB,
            wtA, wtB, rowsA, rowsB, peb, semA, semB):
        wid = lax.axis_index("s") * 2 + lax.axis_index("c")
        wbase = wid * PTS_W
        pltpu.sync_copy(xn_hbm.at[pl.ds(wbase * 3, PTS_W * 3)], xb)

        iota = lax.iota(jnp.int32, 16)
        iota32 = iota * NIN
        iota3 = iota * 3

        def make_grp_a(idxb, colb, wtb):
          def grp_a(g, carry):
            p0 = carry + g * 16
            p3 = iota3 + p0 * 3
            xv = plsc.load_gather(xb, [p3])
            yv = plsc.load_gather(xb, [p3 + 1])
            zv = plsc.load_gather(xb, [p3 + 2])
            gr = g * 16
            for l in range(N_LEVELS):
                res = RES[l]
                rf = float(res)
                px = xv * rf
                py = yv * rf
                pz = zv * rf
                cx0 = px.astype(jnp.int32)
                cy0 = py.astype(jnp.int32)
                cz0 = pz.astype(jnp.int32)
                wx = px - cx0.astype(jnp.float32)
                wy = py - cy0.astype(jnp.float32)
                wz = pz - cz0.astype(jnp.float32)
                cx1 = jnp.minimum(cx0 + 1, res)
                cy1 = jnp.minimum(cy0 + 1, res)
                cz1 = jnp.minimum(cz0 + 1, res)
                if DENSE[l]:
                    r1 = res + 1
                    ax = (cx0, cx1)
                    by = (cy0 * r1, cy1 * r1)
                    tz = (cz0 * (r1 * r1), cz1 * (r1 * r1))
                    idx8 = [ax[dx] + by[dy] + tz[dz] for (dx, dy, dz) in CORNERS]
                else:
                    hx = (cx0.astype(jnp.uint32), cx1.astype(jnp.uint32))
                    hy = (cy0.astype(jnp.uint32) * P1, cy1.astype(jnp.uint32) * P1)
                    hz = (cz0.astype(jnp.uint32) * P2, cz1.astype(jnp.uint32) * P2)
                    idx8 = [
                        ((hx[dx] ^ hy[dy] ^ hz[dz]) & MASK).astype(jnp.int32)
                        for (dx, dy, dz) in CORNERS
                    ]
                wx0 = 1.0 - wx
                wy0 = 1.0 - wy
                wz0 = 1.0 - wz
                wxy = ((wx0 * wy0, wx0 * wy), (wx * wy0, wx * wy))
                wzt = (wz0, wz)
                loff = l << LOG2_T
                wrow = (gr + l) * 128
                for c, (dx, dy, dz) in enumerate(CORNERS):
                    p = idx8[c] + loff
                    sl = pl.ds(wrow + c * 16, 16)
                    idxb[sl] = lax.shift_right_logical(p, 2)
                    colb[sl] = (p & 3) * 2
                    wtb[sl] = wxy[dx][dy] * wzt[dz]
            return carry
          return grp_a

        def make_grp_c(colb, wtb, rowsb):
          def grp_c(g, carry):
            for l in range(N_LEVELS):
                row = g * 16 + l
                wbase_ = row * 128
                f0 = jnp.zeros((16,), jnp.float32)
                f1 = jnp.zeros((16,), jnp.float32)
                for c in range(8):
                    sl = pl.ds(wbase_ + c * 16, 16)
                    wt = wtb[sl]
                    cv = colb[sl]
                    rv = iota + (wbase_ + c * 16)
                    d0 = plsc.load_gather(rowsb, [rv, cv])
                    d1 = plsc.load_gather(rowsb, [rv, cv + 1])
                    f0 = f0 + wt * d0
                    f1 = f1 + wt * d1
                plsc.store_scatter(peb, [iota32 + (g * 16 * NIN + 2 * l)], f0)
                plsc.store_scatter(peb, [iota32 + (g * 16 * NIN + 2 * l + 1)], f1)
            return carry
          return grp_c

        grp_a_A = make_grp_a(idxA, colA, wtA)
        grp_a_B = make_grp_a(idxB, colB, wtB)
        grp_c_A = make_grp_c(colA, wtA, rowsA)
        grp_c_B = make_grp_c(colB, wtB, rowsB)

        def copy_A():
            return pltpu.make_async_copy(tab_hbm.at[idxA], rowsA, semA)

        def copy_B():
            return pltpu.make_async_copy(tab_hbm.at[idxB], rowsB, semB)

        def out_pe(cbase):
            pltpu.sync_copy(peb, pe_hbm.at[pl.ds((wbase + cbase) * NIN, CHUNK * NIN)])

        lax.fori_loop(0, GROUPS, grp_a_A, 0)
        copy_A().start()

        def pair_body(k, carry):
            c0 = (2 * k) * CHUNK
            lax.fori_loop(0, GROUPS, grp_a_B, c0 + CHUNK)
            copy_B().start()
            copy_A().wait()
            lax.fori_loop(0, GROUPS, grp_c_A, 0)
            out_pe(c0)

            @pl.when(k < NPAIR - 1)
            def _():
                lax.fori_loop(0, GROUPS, grp_a_A, c0 + 2 * CHUNK)
                copy_A().start()

            copy_B().wait()
            lax.fori_loop(0, GROUPS, grp_c_B, 0)
            out_pe(c0 + CHUNK)
            return carry

        lax.fori_loop(0, NPAIR, pair_body, 0)

    return enc(xn_t, tab)


def _mlp(pe, W1, b1, W2, b2, W3, b3):
    B = 1024
    grid = (NPTS // B,)

    def body(pe_ref, w1_ref, b1_ref, w2_ref, b2_ref, w3_ref, b3_ref, z_ref, den_ref):
        dn = (((1,), (0,)), ((), ()))
        h = lax.dot_general(pe_ref[...], w1_ref[...], dn,
                            preferred_element_type=jnp.float32)
        h = jnp.maximum(h + b1_ref[...][None, :], 0.0)
        h = lax.dot_general(h, w2_ref[...], dn,
                            preferred_element_type=jnp.float32)
        h = jnp.maximum(h + b2_ref[...][None, :], 0.0)
        z = lax.dot_general(h, w3_ref[...], dn,
                            preferred_element_type=jnp.float32)
        z = z + b3_ref[...][None, :]
        z_ref[...] = z
        z0 = z[:, 0]
        den_ref[...] = jnp.maximum(z0, 0.0) + jnp.log(1.0 + jnp.exp(-jnp.abs(z0)))

    return pl.pallas_call(
        body,
        grid=grid,
        in_specs=[
            pl.BlockSpec((B, NIN), lambda i: (i, 0)),
            pl.BlockSpec((NIN, WIDTH), lambda i: (0, 0)),
            pl.BlockSpec((WIDTH,), lambda i: (0,)),
            pl.BlockSpec((WIDTH, WIDTH), lambda i: (0, 0)),
            pl.BlockSpec((WIDTH,), lambda i: (0,)),
            pl.BlockSpec((WIDTH, NOUT), lambda i: (0, 0)),
            pl.BlockSpec((NOUT,), lambda i: (0,)),
        ],
        out_specs=[
            pl.BlockSpec((B, NOUT), lambda i: (i, 0)),
            pl.BlockSpec((B,), lambda i: (i,)),
        ],
        out_shape=[
            jax.ShapeDtypeStruct((NPTS, NOUT), jnp.float32),
            jax.ShapeDtypeStruct((NPTS,), jnp.float32),
        ],
    )(pe, W1, b1, W2, b2, W3, b3)


def kernel(x, bounding_box, table, W1, b1, W2, b2, W3, b3):
    xn = (x - bounding_box[0]) / (bounding_box[1] - bounding_box[0])
    tab_p = (table.reshape(16, 4096, 128, 2).transpose(0, 1, 3, 2)
             .reshape(-1))
    tab_rm = _sc_relayout(tab_p).reshape(-1, ROWW)
    pe = _sc_encode(xn.reshape(-1), tab_rm).reshape(NPTS, NIN)
    z, density = _mlp(pe, W1, b1, W2, b2, W3, b3)
    return (density, pe, z)


# speedup vs baseline: 8.4222x; 1.1354x over previous
import functools

import jax
import jax.numpy as jnp
import numpy as np
from jax import lax
from jax.experimental import pallas as pl
from jax.experimental.pallas import tpu as pltpu
from jax.experimental.pallas import tpu_sc as plsc

N_LEVELS = 16
F_PER_LEVEL = 2
LOG2_T = 19
TSIZE = 1 << LOG2_T
MASK = np.uint32(TSIZE - 1)
P1 = np.uint32(2654435761)
P2 = np.uint32(805459861)
NPTS = 262144
WIDTH = 128
NOUT = 16
NIN = N_LEVELS * F_PER_LEVEL

RES = [int(np.floor(16 * 1.5 ** l)) for l in range(N_LEVELS)]
DENSE = [(r + 1) ** 3 <= TSIZE for r in RES]
CORNERS = [(dx, dy, dz) for dx in (0, 1) for dy in (0, 1) for dz in (0, 1)]

NW = 32
PTS_W = NPTS // NW
CHUNK = 32
GROUPS = CHUNK // 16
NIDX = CHUNK * 128
ROWW = 8
NCHUNK = PTS_W // CHUNK
NPAIR = NCHUNK // 2
TABF = N_LEVELS * TSIZE * F_PER_LEVEL
RL_CHUNK = 16384


def _sc_relayout(tab_p):
    mesh = plsc.VectorSubcoreMesh(core_axis_name="c", subcore_axis_name="s")
    span = TABF // NW

    @functools.partial(
        pl.kernel,
        out_type=jax.ShapeDtypeStruct((TABF,), jnp.float32),
        mesh=mesh,
        compiler_params=pltpu.CompilerParams(
            needs_layout_passes=False, use_tc_tiling_on_sc=False),
        scratch_types=[
            pltpu.VMEM((RL_CHUNK,), jnp.float32),
            pltpu.VMEM((RL_CHUNK,), jnp.float32),
        ],
    )
    def rl(tab_hbm, out_hbm, inb, outb):
        wid = lax.axis_index("s") * 2 + lax.axis_index("c")
        base = wid * span
        iota2 = lax.iota(jnp.int32, 16) * 2

        def tile_body(t, carry):
            tb = t * 256
            for eb in range(8):
                v0 = inb[pl.ds(tb + eb * 16, 16)]
                v1 = inb[pl.ds(tb + 128 + eb * 16, 16)]
                pos = iota2 + (tb + eb * 32)
                plsc.store_scatter(outb, [pos], v0)
                plsc.store_scatter(outb, [pos + 1], v1)
            return carry

        def chunk_body(ci, carry):
            off = base + ci * RL_CHUNK
            pltpu.sync_copy(tab_hbm.at[pl.ds(off, RL_CHUNK)], inb)
            lax.fori_loop(0, RL_CHUNK // 256, tile_body, 0)
            pltpu.sync_copy(outb, out_hbm.at[pl.ds(off, RL_CHUNK)])
            return carry

        lax.fori_loop(0, span // RL_CHUNK, chunk_body, 0)

    return rl(tab_p)


def _sc_encode(xn_t, tab):
    mesh = plsc.VectorSubcoreMesh(core_axis_name="c", subcore_axis_name="s")

    @functools.partial(
        pl.kernel,
        out_type=jax.ShapeDtypeStruct((NPTS * NIN,), jnp.float32),
        mesh=mesh,
        compiler_params=pltpu.CompilerParams(
            needs_layout_passes=False, use_tc_tiling_on_sc=False),
        scratch_types=[
            pltpu.VMEM((PTS_W * 3,), jnp.float32),
            pltpu.VMEM((NIDX,), jnp.int32),
            pltpu.VMEM((NIDX,), jnp.int32),
            pltpu.VMEM((NIDX,), jnp.int32),
            pltpu.VMEM((NIDX,), jnp.int32),
            pltpu.VMEM((NIDX,), jnp.float32),
            pltpu.VMEM((NIDX,), jnp.float32),
            pltpu.VMEM((NIDX, ROWW), jnp.float32),
            pltpu.VMEM((NIDX, ROWW), jnp.float32),
            pltpu.VMEM((CHUNK * NIN,), jnp.float32),
            pltpu.SemaphoreType.DMA,
            pltpu.SemaphoreType.DMA,
        ],
    )
    def enc(xn_hbm, tab_hbm, pe_hbm, xb, idxA, idxB, colA, colB,
            wtA, wtB, rowsA, rowsB, peb, semA, semB):
        wid = lax.axis_index("s") * 2 + lax.axis_index("c")
        wbase = wid * PTS_W
        pltpu.sync_copy(xn_hbm.at[pl.ds(wbase * 3, PTS_W * 3)], xb)

        iota = lax.iota(jnp.int32, 16)
        iota32 = iota * NIN
        iota3 = iota * 3

        def make_grp_a(idxb, colb, wtb):
          def grp_a(g, carry):
            p0 = carry + g * 16
            p3 = iota3 + p0 * 3
            xv = plsc.load_gather(xb, [p3])
            yv = plsc.load_gather(xb, [p3 + 1])
            zv = plsc.load_gather(xb, [p3 + 2])
            gr = g * 16
            for l in range(N_LEVELS):
                res = RES[l]
                rf = float(res)
                px = xv * rf
                py = yv * rf
                pz = zv * rf
                cx0 = px.astype(jnp.int32)
                cy0 = py.astype(jnp.int32)
                cz0 = pz.astype(jnp.int32)
                wx = px - cx0.astype(jnp.float32)
                wy = py - cy0.astype(jnp.float32)
                wz = pz - cz0.astype(jnp.float32)
                cx1 = jnp.minimum(cx0 + 1, res)
                cy1 = jnp.minimum(cy0 + 1, res)
                cz1 = jnp.minimum(cz0 + 1, res)
                if DENSE[l]:
                    r1 = res + 1
                    ax = (cx0, cx1)
                    by = (cy0 * r1, cy1 * r1)
                    tz = (cz0 * (r1 * r1), cz1 * (r1 * r1))
                    idx8 = [ax[dx] + by[dy] + tz[dz] for (dx, dy, dz) in CORNERS]
                else:
                    hx = (cx0.astype(jnp.uint32), cx1.astype(jnp.uint32))
                    hy = (cy0.astype(jnp.uint32) * P1, cy1.astype(jnp.uint32) * P1)
                    hz = (cz0.astype(jnp.uint32) * P2, cz1.astype(jnp.uint32) * P2)
                    idx8 = [
                        ((hx[dx] ^ hy[dy] ^ hz[dz]) & MASK).astype(jnp.int32)
                        for (dx, dy, dz) in CORNERS
                    ]
                wx0 = 1.0 - wx
                wy0 = 1.0 - wy
                wz0 = 1.0 - wz
                wxy = ((wx0 * wy0, wx0 * wy), (wx * wy0, wx * wy))
                wzt = (wz0, wz)
                loff = l << LOG2_T
                wrow = (gr + l) * 128
                for c, (dx, dy, dz) in enumerate(CORNERS):
                    p = idx8[c] + loff
                    sl = pl.ds(wrow + c * 16, 16)
                    idxb[sl] = lax.shift_right_logical(p, 2)
                    colb[sl] = (p & 3) * 2
                    wtb[sl] = wxy[dx][dy] * wzt[dz]
            return carry
          return grp_a

        def make_grp_c(colb, wtb, rowsb):
          def grp_c(g, carry):
            for l in range(N_LEVELS):
                row = g * 16 + l
                wbase_ = row * 128
                f0 = jnp.zeros((16,), jnp.float32)
                f1 = jnp.zeros((16,), jnp.float32)
                for c in range(8):
                    sl = pl.ds(wbase_ + c * 16, 16)
                    wt = wtb[sl]
                    cv = colb[sl]
                    rv = iota + (wbase_ + c * 16)
                    d0 = plsc.load_gather(rowsb, [rv, cv])
                    d1 = plsc.load_gather(rowsb, [rv, cv + 1])
                    f0 = f0 + wt * d0
                    f1 = f1 + wt * d1
                plsc.store_scatter(peb, [iota32 + (g * 16 * NIN + 2 * l)], f0)
                plsc.store_scatter(peb, [iota32 + (g * 16 * NIN + 2 * l + 1)], f1)
            return carry
          return grp_c

        grp_a_A = make_grp_a(idxA, colA, wtA)
        grp_a_B = make_grp_a(idxB, colB, wtB)
        grp_c_A = make_grp_c(colA, wtA, rowsA)
        grp_c_B = make_grp_c(colB, wtB, rowsB)

        def copy_A():
            return pltpu.make_async_copy(tab_hbm.at[idxA], rowsA, semA)

        def copy_B():
            return pltpu.make_async_copy(tab_hbm.at[idxB], rowsB, semB)

        def out_pe(cbase):
            pltpu.sync_copy(peb, pe_hbm.at[pl.ds((wbase + cbase) * NIN, CHUNK * NIN)])

        lax.fori_loop(0, GROUPS, grp_a_A, 0)
        copy_A().start()

        def pair_body(k, carry):
            c0 = (2 * k) * CHUNK
            lax.fori_loop(0, GROUPS, grp_a_B, c0 + CHUNK)
            copy_B().start()
            copy_A().wait()
            lax.fori_loop(0, GROUPS, grp_c_A, 0)
            out_pe(c0)

            @pl.when(k < NPAIR - 1)
            def _():
                lax.fori_loop(0, GROUPS, grp_a_A, c0 + 2 * CHUNK)
                copy_A().start()

            copy_B().wait()
            lax.fori_loop(0, GROUPS, grp_c_B, 0)
            out_pe(c0 + CHUNK)
            return carry

        lax.fori_loop(0, NPAIR, pair_body, 0)

    return enc(xn_t, tab)


def _mlp(pe, W1, b1, W2, b2, W3, b3):
    B = 2048
    grid = (NPTS // B,)

    def body(pe_ref, w1_ref, b1_ref, w2_ref, b2_ref, w3_ref, b3_ref,
             zt_ref, den_ref):
        dn = (((1,), (0,)), ((), ()))
        h = lax.dot_general(pe_ref[...], w1_ref[...], dn,
                            preferred_element_type=jnp.float32)
        h = jnp.maximum(h + b1_ref[...][None, :], 0.0)
        h = lax.dot_general(h, w2_ref[...], dn,
                            preferred_element_type=jnp.float32)
        h = jnp.maximum(h + b2_ref[...][None, :], 0.0)
        zt = lax.dot_general(w3_ref[...], h, (((0,), (1,)), ((), ())),
                             preferred_element_type=jnp.float32)
        zt = zt + b3_ref[...][:, None]
        zt_ref[...] = zt
        z0 = zt[0, :]
        den_ref[...] = jnp.maximum(z0, 0.0) + jnp.log(1.0 + jnp.exp(-jnp.abs(z0)))

    zt, den = pl.pallas_call(
        body,
        grid=grid,
        in_specs=[
            pl.BlockSpec((B, NIN), lambda i: (i, 0)),
            pl.BlockSpec((NIN, WIDTH), lambda i: (0, 0)),
            pl.BlockSpec((WIDTH,), lambda i: (0,)),
            pl.BlockSpec((WIDTH, WIDTH), lambda i: (0, 0)),
            pl.BlockSpec((WIDTH,), lambda i: (0,)),
            pl.BlockSpec((WIDTH, NOUT), lambda i: (0, 0)),
            pl.BlockSpec((NOUT,), lambda i: (0,)),
        ],
        out_specs=[
            pl.BlockSpec((NOUT, B), lambda i: (0, i)),
            pl.BlockSpec((B,), lambda i: (i,)),
        ],
        out_shape=[
            jax.ShapeDtypeStruct((NOUT, NPTS), jnp.float32),
            jax.ShapeDtypeStruct((NPTS,), jnp.float32),
        ],
    )(pe, W1, b1, W2, b2, W3, b3)
    return zt.T, den


def kernel(x, bounding_box, table, W1, b1, W2, b2, W3, b3):
    xn = (x - bounding_box[0]) / (bounding_box[1] - bounding_box[0])
    tab_p = (table.reshape(16, 4096, 128, 2).transpose(0, 1, 3, 2)
             .reshape(-1))
    tab_rm = _sc_relayout(tab_p).reshape(-1, ROWW)
    pe = _sc_encode(xn.reshape(-1), tab_rm).reshape(NPTS, NIN)
    z, density = _mlp(pe, W1, b1, W2, b2, W3, b3)
    return (density, pe, z)
